# baseline (speedup 1.0000x reference)
import functools

import jax
import jax.numpy as jnp
from jax import lax
from jax.experimental import pallas as pl
from jax.experimental.pallas import tpu as pltpu
from jax.experimental.pallas import tpu_sc as plsc

OUT_DIM = 32

_PI_A = 3.140625
_PI_B = 9.676535897932e-4
_INV_PI = 0.3183098861837907

_S3 = -1.0 / 6.0
_S5 = 1.0 / 120.0
_S7 = -1.0 / 5040.0
_S9 = 1.0 / 362880.0


def _sin(x):
    y = x * _INV_PI
    half = jnp.where(y >= 0.0, 0.5, -0.5)
    k = (y + half).astype(jnp.int32)
    kf = k.astype(jnp.float32)
    r = x - kf * _PI_A
    r = r - kf * _PI_B
    r2 = r * r
    p = _S9
    p = p * r2 + _S7
    p = p * r2 + _S5
    p = p * r2 + _S3
    p = p * r2 + 1.0
    s = r * p
    odd = (k & 1) == 1
    return jnp.where(odd, -s, s)


def kernel(cat_idx, norm_time, w0, b0, w, b):
    N = cat_idx.shape[0]
    NW = 32
    C = 128
    rows_per_worker = N // NW
    chunks = rows_per_worker // C

    wc = jnp.concatenate((w, w0), axis=1)
    bc = jnp.concatenate((b, b0), axis=1)

    mesh = plsc.VectorSubcoreMesh(core_axis_name="c", subcore_axis_name="s")

    @functools.partial(
        pl.kernel,
        mesh=mesh,
        compiler_params=pltpu.CompilerParams(
            needs_layout_passes=False, use_tc_tiling_on_sc=False),
        out_type=jax.ShapeDtypeStruct((N, OUT_DIM), jnp.float32),
        scratch_types=[
            pltpu.VMEM((C,), jnp.int32),
            pltpu.VMEM((C,), jnp.float32),
            pltpu.VMEM((C, OUT_DIM), jnp.float32),
            pltpu.VMEM((C, OUT_DIM), jnp.float32),
            pltpu.VMEM((C, OUT_DIM), jnp.float32),
            pltpu.SemaphoreType.DMA,
        ],
    )
    def sc_kernel(idx_hbm, t_hbm, w_hbm, b_hbm, out_hbm,
                  idx_v, t_v, w_v, b_v, o_v, sem):
        wid = lax.axis_index("s") * 2 + lax.axis_index("c")

        def chunk_body(ci, _):
            base = (wid * chunks + ci) * C
            pltpu.sync_copy(idx_hbm.at[pl.ds(base, C)], idx_v)
            pltpu.sync_copy(t_hbm.at[pl.ds(base, C)], t_v)
            cw = pltpu.async_copy(w_hbm.at[idx_v], w_v, sem)
            cb = pltpu.async_copy(b_hbm.at[idx_v], b_v, sem)
            cw.wait()
            cb.wait()

            def group_body(g, _):
                rows = lax.iota(jnp.int32, 16) + g * 16
                tv = t_v[pl.ds(pl.multiple_of(g * 16, 16), 16)]

                def col_body(j, _):
                    jj = jnp.full((16,), j, jnp.int32)
                    wv = plsc.load_gather(w_v, [rows, jj])
                    bv = plsc.load_gather(b_v, [rows, jj])
                    a = tv * wv + bv
                    s = jnp.where(jj < OUT_DIM - 1, _sin(a), a)
                    plsc.store_scatter(o_v, [rows, jj], s)
                    return 0

                lax.fori_loop(0, OUT_DIM, col_body, 0)
                return 0

            lax.fori_loop(0, C // 16, group_body, 0)
            pltpu.sync_copy(o_v, out_hbm.at[pl.ds(base, C)])
            return 0

        lax.fori_loop(0, chunks, chunk_body, 0)

    return sc_kernel(cat_idx, norm_time, wc, bc)


# speedup vs baseline: 1.1179x; 1.1179x over previous
import functools

import jax
import jax.numpy as jnp
from jax import lax
from jax.experimental import pallas as pl
from jax.experimental.pallas import tpu as pltpu
from jax.experimental.pallas import tpu_sc as plsc

OUT_DIM = 32

_PI_A = 3.140625
_PI_B = 9.676535897932e-4
_INV_PI = 0.3183098861837907

_S3 = -1.0 / 6.0
_S5 = 1.0 / 120.0
_S7 = -1.0 / 5040.0
_S9 = 1.0 / 362880.0


def _sin(x):
    y = x * _INV_PI
    half = jnp.where(y >= 0.0, 0.5, -0.5)
    k = (y + half).astype(jnp.int32)
    kf = k.astype(jnp.float32)
    r = x - kf * _PI_A
    r = r - kf * _PI_B
    r2 = r * r
    p = _S9
    p = p * r2 + _S7
    p = p * r2 + _S5
    p = p * r2 + _S3
    p = p * r2 + 1.0
    s = r * p
    odd = (k & 1) == 1
    return jnp.where(odd, -s, s)


def kernel(cat_idx, norm_time, w0, b0, w, b):
    N = cat_idx.shape[0]
    NW = 32
    C = 128
    rpw = N // NW
    chunks = rpw // C
    assert chunks % 2 == 0

    wc = jnp.concatenate((w, w0), axis=1)
    bc = jnp.concatenate((b, b0), axis=1)

    mesh = plsc.VectorSubcoreMesh(core_axis_name="c", subcore_axis_name="s")

    @functools.partial(
        pl.kernel,
        mesh=mesh,
        compiler_params=pltpu.CompilerParams(
            needs_layout_passes=False, use_tc_tiling_on_sc=False),
        out_type=jax.ShapeDtypeStruct((N, OUT_DIM), jnp.float32),
        scratch_types=[
            pltpu.VMEM((rpw,), jnp.int32),
            pltpu.VMEM((rpw,), jnp.float32),
            pltpu.VMEM((2, C, OUT_DIM), jnp.float32),
            pltpu.VMEM((2, C, OUT_DIM), jnp.float32),
            pltpu.VMEM((2, C, OUT_DIM), jnp.float32),
            pltpu.SemaphoreType.DMA,
            pltpu.SemaphoreType.DMA,
            pltpu.SemaphoreType.DMA,
            pltpu.SemaphoreType.DMA,
        ],
    )
    def sc_kernel(idx_hbm, t_hbm, w_hbm, b_hbm, out_hbm,
                  idx_a, t_a, w_v, b_v, o_v, gsem0, gsem1, osem0, osem1):
        wid = lax.axis_index("s") * 2 + lax.axis_index("c")
        wbase = wid * rpw
        pltpu.sync_copy(idx_hbm.at[pl.ds(wbase, rpw)], idx_a)
        pltpu.sync_copy(t_hbm.at[pl.ds(wbase, rpw)], t_a)
        gsems = (gsem0, gsem1)
        osems = (osem0, osem1)

        def issue_gathers(ci, buf):
            isl = idx_a.at[pl.ds(ci * C, C)]
            pltpu.async_copy(w_hbm.at[isl], w_v.at[buf], gsems[buf])
            pltpu.async_copy(b_hbm.at[isl], b_v.at[buf], gsems[buf])

        def wait_gathers(buf):
            pltpu.make_async_copy(w_hbm.at[pl.ds(0, C)], w_v.at[buf],
                                  gsems[buf]).wait()
            pltpu.make_async_copy(b_hbm.at[pl.ds(0, C)], b_v.at[buf],
                                  gsems[buf]).wait()

        def wait_out(buf):
            pltpu.make_async_copy(o_v.at[buf], out_hbm.at[pl.ds(0, C)],
                                  osems[buf]).wait()

        issue_gathers(0, 0)

        def pair_body(p, _):
            for buf in (0, 1):
                ci = p * 2 + buf
                if buf == 0:
                    issue_gathers(ci + 1, 1)
                else:
                    @pl.when(p < chunks // 2 - 1)
                    def _():
                        issue_gathers(ci + 1, 0)
                wait_gathers(buf)

                @pl.when(ci >= 2)
                def _():
                    wait_out(buf)

                wvb = w_v.at[buf]
                bvb = b_v.at[buf]
                ovb = o_v.at[buf]

                def group_body(g, _):
                    rows = lax.iota(jnp.int32, 16) + g * 16
                    tv = t_a[pl.ds(pl.multiple_of(ci * C + g * 16, 16), 16)]
                    for j in range(OUT_DIM):
                        jj = jnp.full((16,), j, jnp.int32)
                        wv = plsc.load_gather(wvb, [rows, jj])
                        bv = plsc.load_gather(bvb, [rows, jj])
                        a = tv * wv + bv
                        s = _sin(a) if j < OUT_DIM - 1 else a
                        plsc.store_scatter(ovb, [rows, jj], s)
                    return 0

                lax.fori_loop(0, C // 16, group_body, 0)
                pltpu.async_copy(
                    ovb, out_hbm.at[pl.ds(wbase + ci * C, C)], osems[buf])
            return 0

        lax.fori_loop(0, chunks // 2, pair_body, 0)
        wait_out(0)
        wait_out(1)

    return sc_kernel(cat_idx, norm_time, wc, bc)


# speedup vs baseline: 1.7183x; 1.5371x over previous
import functools

import jax
import jax.numpy as jnp
from jax import lax
from jax.experimental import pallas as pl
from jax.experimental.pallas import tpu as pltpu
from jax.experimental.pallas import tpu_sc as plsc

OUT_DIM = 32

_PI = 3.14159265358979
_INV_PI = 0.3183098861837907
_MAGIC = 12582912.0

_S3 = -1.0 / 6.0
_S5 = 1.0 / 120.0
_S7 = -1.0 / 5040.0
_S9 = 1.0 / 362880.0


def _sin(x):
    y = x * _INV_PI
    tmp = y + _MAGIC
    kf = tmp - _MAGIC
    r = x - kf * _PI
    r2 = r * r
    p = _S9
    p = p * r2 + _S7
    p = p * r2 + _S5
    p = p * r2 + _S3
    p = p * r2 + 1.0
    s = r * p
    sgn = (plsc.bitcast(tmp, jnp.int32) & 1) << 31
    return plsc.bitcast(plsc.bitcast(s, jnp.int32) ^ sgn, jnp.float32)


def kernel(cat_idx, norm_time, w0, b0, w, b):
    N = cat_idx.shape[0]
    NW = 32
    C = 128
    rpw = N // NW
    chunks = rpw // C
    assert chunks % 2 == 0

    wc = jnp.concatenate((w, w0), axis=1)
    bc = jnp.concatenate((b, b0), axis=1)

    mesh = plsc.VectorSubcoreMesh(core_axis_name="c", subcore_axis_name="s")

    @functools.partial(
        pl.kernel,
        mesh=mesh,
        compiler_params=pltpu.CompilerParams(
            needs_layout_passes=False, use_tc_tiling_on_sc=False),
        out_type=jax.ShapeDtypeStruct((N * OUT_DIM,), jnp.float32),
        scratch_types=[
            pltpu.VMEM((rpw,), jnp.int32),
            pltpu.VMEM((rpw,), jnp.float32),
            pltpu.VMEM((2, C, OUT_DIM), jnp.float32),
            pltpu.VMEM((2, C, OUT_DIM), jnp.float32),
            pltpu.VMEM((2, C * OUT_DIM), jnp.float32),
            pltpu.SemaphoreType.DMA,
            pltpu.SemaphoreType.DMA,
            pltpu.SemaphoreType.DMA,
            pltpu.SemaphoreType.DMA,
        ],
    )
    def sc_kernel(idx_hbm, t_hbm, w_hbm, b_hbm, out_hbm,
                  idx_a, t_a, w_v, b_v, o_v, gsem0, gsem1, osem0, osem1):
        wid = lax.axis_index("s") * 2 + lax.axis_index("c")
        wbase = wid * rpw
        pltpu.sync_copy(idx_hbm.at[pl.ds(wbase, rpw)], idx_a)
        pltpu.sync_copy(t_hbm.at[pl.ds(wbase, rpw)], t_a)
        gsems = (gsem0, gsem1)
        osems = (osem0, osem1)

        def issue_gathers(ci, buf):
            isl = idx_a.at[pl.ds(ci * C, C)]
            pltpu.async_copy(w_hbm.at[isl], w_v.at[buf], gsems[buf])
            pltpu.async_copy(b_hbm.at[isl], b_v.at[buf], gsems[buf])

        def wait_gathers(buf):
            pltpu.make_async_copy(w_hbm.at[pl.ds(0, C)], w_v.at[buf],
                                  gsems[buf]).wait()
            pltpu.make_async_copy(b_hbm.at[pl.ds(0, C)], b_v.at[buf],
                                  gsems[buf]).wait()

        def wait_out(buf):
            pltpu.make_async_copy(o_v.at[buf], out_hbm.at[pl.ds(0, C * OUT_DIM)],
                                  osems[buf]).wait()

        issue_gathers(0, 0)

        def pair_body(p, _):
            for buf in (0, 1):
                ci = p * 2 + buf
                if buf == 0:
                    issue_gathers(ci + 1, 1)
                else:
                    @pl.when(p < chunks // 2 - 1)
                    def _():
                        issue_gathers(ci + 1, 0)
                wait_gathers(buf)

                @pl.when(ci >= 2)
                def _():
                    wait_out(buf)

                wvb = w_v.at[buf]
                bvb = b_v.at[buf]
                ovb = o_v.at[buf]
                lanes32 = lax.iota(jnp.int32, 16) * OUT_DIM

                @plsc.parallel_loop(0, (C // 16) * (OUT_DIM - 1), unroll=8)
                def _(i):
                    g = i & 7
                    j = i >> 3
                    rows = lax.iota(jnp.int32, 16) + g * 16
                    tv = t_a[pl.ds(pl.multiple_of(ci * C + g * 16, 16), 16)]
                    jj = jnp.full((16,), j, jnp.int32)
                    wv = plsc.load_gather(wvb, [rows, jj])
                    bv = plsc.load_gather(bvb, [rows, jj])
                    s = _sin(tv * wv + bv)
                    plsc.store_scatter(ovb, [lanes32 + (g * (16 * OUT_DIM) + j)], s)

                @plsc.parallel_loop(0, C // 16, unroll=4)
                def _(g):
                    rows = lax.iota(jnp.int32, 16) + g * 16
                    tv = t_a[pl.ds(pl.multiple_of(ci * C + g * 16, 16), 16)]
                    jj = jnp.full((16,), OUT_DIM - 1, jnp.int32)
                    wv = plsc.load_gather(wvb, [rows, jj])
                    bv = plsc.load_gather(bvb, [rows, jj])
                    lin = tv * wv + bv
                    plsc.store_scatter(
                        ovb, [lanes32 + (g * (16 * OUT_DIM) + (OUT_DIM - 1))], lin)

                pltpu.async_copy(
                    ovb, out_hbm.at[pl.ds((wbase + ci * C) * OUT_DIM, C * OUT_DIM)],
                    osems[buf])
            return 0

        lax.fori_loop(0, chunks // 2, pair_body, 0)
        wait_out(0)
        wait_out(1)

    out_flat = sc_kernel(cat_idx, norm_time, wc, bc)
    return out_flat.reshape(N, OUT_DIM)
